# Initial kernel scaffold; baseline (speedup 1.0000x reference)
import functools
import jax
import jax.numpy as jnp
from jax import lax
from jax.experimental import pallas as pl
from jax.experimental.pallas import tpu as pltpu
from jax.experimental.pallas import tpu_sc as plsc

N_U = 10000
N_V = 10000
E = 320000
D_IN = 128
D_OUT = 128
NS = 4
D_SUB = D_OUT // NS
E_BLK = E // NS
RAW_C = 125
C = 128
CHUNKS = E_BLK // RAW_C
N_TILES = 16
CPT = CHUNKS // N_TILES
ROWS_PT = N_U // N_TILES


def _tc_matmul(xu_ref, xv_ref, w_ref, out_ref):
    d = pl.program_id(0)

    @pl.when(d == 0)
    def _():
        out_ref[0, 0] = jnp.dot(xv_ref[...], w_ref[...],
                                preferred_element_type=jnp.float32)

    @pl.when(d == 1)
    def _():
        out_ref[0, 0] = jnp.dot(xu_ref[...], w_ref[...],
                                preferred_element_type=jnp.float32)


_SC_MESH = plsc.VectorSubcoreMesh(core_axis_name="c", subcore_axis_name="s")


@functools.partial(
    pl.kernel,
    out_type=jax.ShapeDtypeStruct((2, NS, N_U, D_SUB), jnp.float32),
    mesh=_SC_MESH,
    scratch_types=[
        pltpu.VMEM_SHARED((NS, N_U, D_SUB), jnp.float32),
        pltpu.VMEM((CPT, C), jnp.int32),
        pltpu.VMEM((CPT, C), jnp.int32),
        pltpu.VMEM((CPT, C), jnp.float32),
        pltpu.VMEM((2, C, D_SUB), jnp.float32),
        pltpu.VMEM((ROWS_PT, D_SUB), jnp.float32),
        pltpu.SemaphoreType.DMA((2,)),
    ],
)
def _sc_aggregate(t_hbm, g_hbm, s_hbm, v_hbm, out_hbm,
                  acc, gidx, sidx, vals, rows, buf, sems):
    d = lax.axis_index("c")
    t = lax.axis_index("s")

    zvec = jnp.zeros((16,), jnp.float32)

    def _zero(j, carry):
        buf[j, pl.ds(0, 16)] = zvec
        buf[j, pl.ds(16, 16)] = zvec
        return carry
    lax.fori_loop(0, ROWS_PT, _zero, 0)
    for i in range(NS):
        pltpu.sync_copy(buf, acc.at[i].at[pl.ds(t * ROWS_PT, ROWS_PT)])
    plsc.subcore_barrier()

    for i in range(NS):
        pltpu.sync_copy(g_hbm.at[d].at[i].at[pl.ds(t * CPT, CPT)], gidx)
        pltpu.sync_copy(s_hbm.at[d].at[i].at[pl.ds(t * CPT, CPT)], sidx)
        pltpu.sync_copy(v_hbm.at[d].at[i].at[pl.ds(t * CPT, CPT)], vals)

        table = t_hbm.at[d].at[i]
        pltpu.async_copy(table.at[gidx.at[0]], rows.at[0], sems.at[0])

        def _chunk(g, carry):
            b = lax.rem(g, 2)
            nb = lax.rem(g + 1, 2)

            @pl.when(g + 1 < CPT)
            def _():
                pltpu.async_copy(table.at[gidx.at[g + 1]], rows.at[nb],
                                 sems.at[nb])

            pltpu.make_async_copy(table.at[gidx.at[g]], rows.at[b],
                                  sems.at[b]).wait()

            def _scale(e, c2):
                s = vals[g, e]
                rows[b, e, pl.ds(0, 16)] = rows[b, e, pl.ds(0, 16)] * s
                rows[b, e, pl.ds(16, 16)] = rows[b, e, pl.ds(16, 16)] * s
                return c2
            lax.fori_loop(0, C, _scale, 0)

            pltpu.sync_copy(rows.at[b], acc.at[i].at[sidx.at[g]], add=True)
            return carry
        lax.fori_loop(0, CPT, _chunk, 0)

    plsc.subcore_barrier()

    for i in range(NS):
        pltpu.sync_copy(acc.at[i].at[pl.ds(t * ROWS_PT, ROWS_PT)], buf)

        def _relu(j, carry):
            buf[j, pl.ds(0, 16)] = jnp.maximum(buf[j, pl.ds(0, 16)], zvec)
            buf[j, pl.ds(16, 16)] = jnp.maximum(buf[j, pl.ds(16, 16)], zvec)
            return carry
        lax.fori_loop(0, ROWS_PT, _relu, 0)
        pltpu.sync_copy(buf, out_hbm.at[d].at[i].at[pl.ds(t * ROWS_PT, ROWS_PT)])


def _pad_chunks(a):
    a = a.reshape(2, NS, CHUNKS, RAW_C)
    return jnp.pad(a, ((0, 0), (0, 0), (0, 0), (0, C - RAW_C)))


def kernel(x_u, x_v, edge_u, edge_v, sup_vals, sup_t_vals, W):
    T = pl.pallas_call(
        _tc_matmul,
        grid=(2, NS),
        in_specs=[
            pl.BlockSpec((N_U, D_IN), lambda d, i: (0, 0)),
            pl.BlockSpec((N_V, D_IN), lambda d, i: (0, 0)),
            pl.BlockSpec((D_IN, D_SUB), lambda d, i: (0, i)),
        ],
        out_specs=pl.BlockSpec((1, 1, N_U, D_SUB), lambda d, i: (d, i, 0, 0)),
        out_shape=jax.ShapeDtypeStruct((2, NS, N_U, D_SUB), jnp.float32),
    )(x_u, x_v, W)

    G = _pad_chunks(jnp.stack([edge_v, edge_u]))
    S = _pad_chunks(jnp.stack([edge_u, edge_v]))
    V = _pad_chunks(jnp.stack([sup_vals, sup_t_vals]))

    Z = _sc_aggregate(T, G, S, V)

    u_out = Z[0].transpose(1, 0, 2).reshape(N_U, D_OUT)
    v_out = Z[1].transpose(1, 0, 2).reshape(N_V, D_OUT)
    return (u_out, v_out)


# speedup vs baseline: 9.7596x; 9.7596x over previous
import functools
import jax
import jax.numpy as jnp
from jax import lax
from jax.experimental import pallas as pl
from jax.experimental.pallas import tpu as pltpu
from jax.experimental.pallas import tpu_sc as plsc

N_U = 10000
N_V = 10000
E = 320000
D_IN = 128
D_OUT = 128
NS = 4
D_SUB = D_OUT // NS
E_BLK = E // NS
RAW_C = 125
C = 128
CHUNKS = E_BLK // RAW_C
N_TILES = 16
CPT = CHUNKS // N_TILES
NR0 = 624
NRL = 640


def _tc_matmul(xu_ref, xv_ref, w_ref, out_ref):
    d = pl.program_id(0)

    @pl.when(d == 0)
    def _():
        out_ref[0, 0] = jnp.dot(xv_ref[...], w_ref[0],
                                preferred_element_type=jnp.float32)

    @pl.when(d == 1)
    def _():
        out_ref[0, 0] = jnp.dot(xu_ref[...], w_ref[0],
                                preferred_element_type=jnp.float32)


_SC_MESH = plsc.VectorSubcoreMesh(core_axis_name="c", subcore_axis_name="s")


@functools.partial(
    pl.kernel,
    out_type=jax.ShapeDtypeStruct((2, NS, N_U, D_SUB), jnp.float32),
    mesh=_SC_MESH,
    scratch_types=[
        pltpu.VMEM_SHARED((NS, N_U, D_SUB), jnp.float32),
        pltpu.VMEM((CPT, C), jnp.int32),
        pltpu.VMEM((CPT, C), jnp.int32),
        pltpu.VMEM((CPT, C), jnp.float32),
        pltpu.VMEM((2, C, D_SUB), jnp.float32),
        pltpu.VMEM((NRL, D_SUB), jnp.float32),
        pltpu.SemaphoreType.DMA((2,)),
    ],
    compiler_params=pltpu.CompilerParams(use_tc_tiling_on_sc=False),
)
def _sc_aggregate(t_hbm, g_hbm, s_hbm, v_hbm, out_hbm,
                  acc, gidx, sidx, vals, rows, buf, sems):
    d = lax.axis_index("c")
    t = lax.axis_index("s")

    zvec = jnp.zeros((16,), jnp.float32)
    start = t * NR0
    nrows = jnp.where(t == N_TILES - 1, NRL, NR0)

    def _zero(j, carry):
        buf[j, pl.ds(0, 16)] = zvec
        buf[j, pl.ds(16, 16)] = zvec
        return carry
    lax.fori_loop(0, NRL, _zero, 0)
    for i in range(NS):
        @pl.when(t < N_TILES - 1)
        def _():
            pltpu.sync_copy(buf.at[pl.ds(0, NR0)],
                            acc.at[i].at[pl.ds(start, NR0)])

        @pl.when(t == N_TILES - 1)
        def _():
            pltpu.sync_copy(buf, acc.at[i].at[pl.ds(start, NRL)])
    plsc.subcore_barrier()

    for i in range(NS):
        pltpu.sync_copy(g_hbm.at[d].at[i].at[pl.ds(t * CPT, CPT)], gidx)
        pltpu.sync_copy(s_hbm.at[d].at[i].at[pl.ds(t * CPT, CPT)], sidx)
        pltpu.sync_copy(v_hbm.at[d].at[i].at[pl.ds(t * CPT, CPT)], vals)

        table = t_hbm.at[d].at[i]
        pltpu.async_copy(table.at[gidx.at[0]], rows.at[0], sems.at[0])

        def _chunk(g, carry):
            b = lax.rem(g, 2)
            nb = lax.rem(g + 1, 2)

            @pl.when(g + 1 < CPT)
            def _():
                pltpu.async_copy(table.at[gidx.at[g + 1]], rows.at[nb],
                                 sems.at[nb])

            pltpu.make_async_copy(table.at[gidx.at[g]], rows.at[b],
                                  sems.at[b]).wait()

            def _scale(q, c2):
                svec = vals[g, pl.ds(q * 16, 16)]
                for l in range(16):
                    e = q * 16 + l
                    s = svec[l]
                    rows[b, e, pl.ds(0, 16)] = rows[b, e, pl.ds(0, 16)] * s
                    rows[b, e, pl.ds(16, 16)] = rows[b, e, pl.ds(16, 16)] * s
                return c2
            lax.fori_loop(0, C // 16, _scale, 0)

            pltpu.sync_copy(rows.at[b], acc.at[i].at[sidx.at[g]], add=True)
            return carry
        lax.fori_loop(0, CPT, _chunk, 0)

    plsc.subcore_barrier()

    for i in range(NS):
        @pl.when(t < N_TILES - 1)
        def _():
            pltpu.sync_copy(acc.at[i].at[pl.ds(start, NR0)],
                            buf.at[pl.ds(0, NR0)])

        @pl.when(t == N_TILES - 1)
        def _():
            pltpu.sync_copy(acc.at[i].at[pl.ds(start, NRL)], buf)

        def _relu(j, carry):
            buf[j, pl.ds(0, 16)] = jnp.maximum(buf[j, pl.ds(0, 16)], zvec)
            buf[j, pl.ds(16, 16)] = jnp.maximum(buf[j, pl.ds(16, 16)], zvec)
            return carry
        lax.fori_loop(0, nrows, _relu, 0)

        @pl.when(t < N_TILES - 1)
        def _():
            pltpu.sync_copy(buf.at[pl.ds(0, NR0)],
                            out_hbm.at[d].at[i].at[pl.ds(start, NR0)])

        @pl.when(t == N_TILES - 1)
        def _():
            pltpu.sync_copy(buf, out_hbm.at[d].at[i].at[pl.ds(start, NRL)])


def _pad_chunks(a):
    a = a.reshape(2, NS, CHUNKS, RAW_C)
    return jnp.pad(a, ((0, 0), (0, 0), (0, 0), (0, C - RAW_C)))


def kernel(x_u, x_v, edge_u, edge_v, sup_vals, sup_t_vals, W):
    T = pl.pallas_call(
        _tc_matmul,
        grid=(2, NS),
        in_specs=[
            pl.BlockSpec((N_U, D_IN), lambda d, i: (0, 0)),
            pl.BlockSpec((N_V, D_IN), lambda d, i: (0, 0)),
            pl.BlockSpec((1, D_IN, D_SUB), lambda d, i: (i, 0, 0)),
        ],
        out_specs=pl.BlockSpec((1, 1, N_U, D_SUB), lambda d, i: (d, i, 0, 0)),
        out_shape=jax.ShapeDtypeStruct((2, NS, N_U, D_SUB), jnp.float32),
    )(x_u, x_v, jnp.transpose(W.reshape(D_IN, NS, D_SUB), (1, 0, 2)))

    G = _pad_chunks(jnp.stack([edge_v, edge_u]))
    S = _pad_chunks(jnp.stack([edge_u, edge_v]))
    V = _pad_chunks(jnp.stack([sup_vals, sup_t_vals]))

    Z = _sc_aggregate(T, G, S, V)

    u_out = Z[0].transpose(1, 0, 2).reshape(N_U, D_OUT)
    v_out = Z[1].transpose(1, 0, 2).reshape(N_V, D_OUT)
    return (u_out, v_out)


# speedup vs baseline: 13.3375x; 1.3666x over previous
import functools
import jax
import jax.numpy as jnp
from jax import lax
from jax.experimental import pallas as pl
from jax.experimental.pallas import tpu as pltpu
from jax.experimental.pallas import tpu_sc as plsc

N_U = 10000
N_V = 10000
E = 320000
D_IN = 128
D_OUT = 128
NS = 4
D_SUB = D_OUT // NS
E_BLK = E // NS
RAW_C = 125
C = 128
CHUNKS = E_BLK // RAW_C
N_TILES = 16
CPT = CHUNKS // N_TILES
NR0 = 624
NRL = 640


def _tc_matmul(xu_ref, xv_ref, w_ref, out_ref):
    d = pl.program_id(0)

    @pl.when(d == 0)
    def _():
        out_ref[0, 0] = jnp.dot(xv_ref[...], w_ref[0],
                                preferred_element_type=jnp.float32)

    @pl.when(d == 1)
    def _():
        out_ref[0, 0] = jnp.dot(xu_ref[...], w_ref[0],
                                preferred_element_type=jnp.float32)


_SC_MESH = plsc.VectorSubcoreMesh(core_axis_name="c", subcore_axis_name="s")


@functools.partial(
    pl.kernel,
    out_type=(jax.ShapeDtypeStruct((N_U, D_OUT), jnp.float32),
              jax.ShapeDtypeStruct((N_V, D_OUT), jnp.float32)),
    mesh=_SC_MESH,
    scratch_types=[
        pltpu.VMEM_SHARED((NS, N_U, D_SUB), jnp.float32),
        pltpu.VMEM((CPT, C), jnp.int32),
        pltpu.VMEM((CPT, C), jnp.int32),
        pltpu.VMEM((CPT, C), jnp.float32),
        pltpu.VMEM((2, C, D_SUB), jnp.float32),
        pltpu.VMEM((NRL, D_SUB), jnp.float32),
        pltpu.SemaphoreType.DMA((2,)),
    ],
    compiler_params=pltpu.CompilerParams(use_tc_tiling_on_sc=False),
)
def _sc_aggregate(t_hbm, eu_hbm, ev_hbm, s0_hbm, s1_hbm, outu_hbm, outv_hbm,
                  acc, gidx, sidx, vals, rows, buf, sems):
    d = lax.axis_index("c")
    t = lax.axis_index("s")

    zvec = jnp.zeros((16,), jnp.float32)
    start = t * NR0
    nrows = jnp.where(t == N_TILES - 1, NRL, NR0)

    def _zero(j, carry):
        buf[j, pl.ds(0, 16)] = zvec
        buf[j, pl.ds(16, 16)] = zvec
        return carry
    lax.fori_loop(0, NRL, _zero, 0)
    for i in range(NS):
        @pl.when(t < N_TILES - 1)
        def _():
            pltpu.sync_copy(buf.at[pl.ds(0, NR0)],
                            acc.at[i].at[pl.ds(start, NR0)])

        @pl.when(t == N_TILES - 1)
        def _():
            pltpu.sync_copy(buf, acc.at[i].at[pl.ds(start, NRL)])
    plsc.subcore_barrier()

    for i in range(NS):
        @pl.when(d == 0)
        def _():
            pltpu.sync_copy(ev_hbm.at[i].at[pl.ds(t * CPT, CPT)], gidx)
            pltpu.sync_copy(eu_hbm.at[i].at[pl.ds(t * CPT, CPT)], sidx)
            pltpu.sync_copy(s0_hbm.at[i].at[pl.ds(t * CPT, CPT)], vals)

        @pl.when(d == 1)
        def _():
            pltpu.sync_copy(eu_hbm.at[i].at[pl.ds(t * CPT, CPT)], gidx)
            pltpu.sync_copy(ev_hbm.at[i].at[pl.ds(t * CPT, CPT)], sidx)
            pltpu.sync_copy(s1_hbm.at[i].at[pl.ds(t * CPT, CPT)], vals)

        table = t_hbm.at[d].at[i]
        pltpu.async_copy(table.at[gidx.at[0]], rows.at[0], sems.at[0])

        def _chunk(g, carry):
            b = lax.rem(g, 2)
            nb = lax.rem(g + 1, 2)

            @pl.when(g + 1 < CPT)
            def _():
                pltpu.async_copy(table.at[gidx.at[g + 1]], rows.at[nb],
                                 sems.at[nb])

            pltpu.make_async_copy(table.at[gidx.at[g]], rows.at[b],
                                  sems.at[b]).wait()

            def _scale(q, c2):
                svec = vals[g, pl.ds(q * 16, 16)]
                for l in range(16):
                    e = q * 16 + l
                    s = svec[l]
                    rows[b, e, pl.ds(0, 16)] = rows[b, e, pl.ds(0, 16)] * s
                    rows[b, e, pl.ds(16, 16)] = rows[b, e, pl.ds(16, 16)] * s
                return c2
            lax.fori_loop(0, C // 16, _scale, 0)

            pltpu.sync_copy(rows.at[b], acc.at[i].at[sidx.at[g]], add=True)
            return carry
        lax.fori_loop(0, CPT, _chunk, 0)

    plsc.subcore_barrier()

    for i in range(NS):
        @pl.when(t < N_TILES - 1)
        def _():
            pltpu.sync_copy(acc.at[i].at[pl.ds(start, NR0)],
                            buf.at[pl.ds(0, NR0)])

        @pl.when(t == N_TILES - 1)
        def _():
            pltpu.sync_copy(acc.at[i].at[pl.ds(start, NRL)], buf)

        def _relu(j, carry):
            buf[j, pl.ds(0, 16)] = jnp.maximum(buf[j, pl.ds(0, 16)], zvec)
            buf[j, pl.ds(16, 16)] = jnp.maximum(buf[j, pl.ds(16, 16)], zvec)
            return carry
        lax.fori_loop(0, nrows, _relu, 0)

        col = pl.ds(i * D_SUB, D_SUB)

        @pl.when((d == 0) & (t < N_TILES - 1))
        def _():
            pltpu.sync_copy(buf.at[pl.ds(0, NR0)],
                            outu_hbm.at[pl.ds(start, NR0), col])

        @pl.when((d == 0) & (t == N_TILES - 1))
        def _():
            pltpu.sync_copy(buf, outu_hbm.at[pl.ds(start, NRL), col])

        @pl.when((d == 1) & (t < N_TILES - 1))
        def _():
            pltpu.sync_copy(buf.at[pl.ds(0, NR0)],
                            outv_hbm.at[pl.ds(start, NR0), col])

        @pl.when((d == 1) & (t == N_TILES - 1))
        def _():
            pltpu.sync_copy(buf, outv_hbm.at[pl.ds(start, NRL), col])


def _pad_chunks(a):
    a = a.reshape(NS, CHUNKS, RAW_C)
    return jnp.pad(a, ((0, 0), (0, 0), (0, C - RAW_C)))


def kernel(x_u, x_v, edge_u, edge_v, sup_vals, sup_t_vals, W):
    T = pl.pallas_call(
        _tc_matmul,
        grid=(2, NS),
        in_specs=[
            pl.BlockSpec((N_U, D_IN), lambda d, i: (0, 0)),
            pl.BlockSpec((N_V, D_IN), lambda d, i: (0, 0)),
            pl.BlockSpec((1, D_IN, D_SUB), lambda d, i: (i, 0, 0)),
        ],
        out_specs=pl.BlockSpec((1, 1, N_U, D_SUB), lambda d, i: (d, i, 0, 0)),
        out_shape=jax.ShapeDtypeStruct((2, NS, N_U, D_SUB), jnp.float32),
    )(x_u, x_v, jnp.transpose(W.reshape(D_IN, NS, D_SUB), (1, 0, 2)))

    eu = _pad_chunks(edge_u)
    ev = _pad_chunks(edge_v)
    s0 = _pad_chunks(sup_vals)
    s1 = _pad_chunks(sup_t_vals)

    u_out, v_out = _sc_aggregate(T, eu, ev, s0, s1)
    return (u_out, v_out)


# speedup vs baseline: 15.6851x; 1.1760x over previous
import functools
import jax
import jax.numpy as jnp
from jax import lax
from jax.experimental import pallas as pl
from jax.experimental.pallas import tpu as pltpu
from jax.experimental.pallas import tpu_sc as plsc

N_U = 10000
N_V = 10000
E = 320000
D_IN = 128
D_OUT = 128
NS = 4
D_SUB = D_OUT // NS
E_BLK = E // NS
C = 128
CHUNKS = E_BLK // C
N_TILES = 16
CP0 = 39
CPL = 40
NR0 = 624
NRL = 640


def _tc_matmul(xu_ref, xv_ref, w_ref, out_ref):
    d = pl.program_id(0)

    @pl.when(d == 0)
    def _():
        out_ref[0, 0] = jnp.dot(xv_ref[...], w_ref[0],
                                preferred_element_type=jnp.float32)

    @pl.when(d == 1)
    def _():
        out_ref[0, 0] = jnp.dot(xu_ref[...], w_ref[0],
                                preferred_element_type=jnp.float32)


_SC_MESH = plsc.VectorSubcoreMesh(core_axis_name="c", subcore_axis_name="s")


@functools.partial(
    pl.kernel,
    out_type=(jax.ShapeDtypeStruct((N_U, D_OUT), jnp.float32),
              jax.ShapeDtypeStruct((N_V, D_OUT), jnp.float32)),
    mesh=_SC_MESH,
    scratch_types=[
        pltpu.VMEM_SHARED((NS, N_U, D_SUB), jnp.float32),
        pltpu.VMEM((CPL, C), jnp.int32),
        pltpu.VMEM((CPL, C), jnp.int32),
        pltpu.VMEM((CPL, C), jnp.float32),
        pltpu.VMEM((2, C, D_SUB), jnp.float32),
        pltpu.VMEM((NRL, D_SUB), jnp.float32),
        pltpu.SemaphoreType.DMA((2,)),
        pltpu.SemaphoreType.DMA((2,)),
    ],
    compiler_params=pltpu.CompilerParams(use_tc_tiling_on_sc=False),
)
def _sc_aggregate(t_hbm, eu_hbm, ev_hbm, s0_hbm, s1_hbm, outu_hbm, outv_hbm,
                  acc, gidx, sidx, vals, rows, buf, sems, ssems):
    d = lax.axis_index("c")
    t = lax.axis_index("s")

    zvec = jnp.zeros((16,), jnp.float32)
    start = t * NR0
    nrows = jnp.where(t == N_TILES - 1, NRL, NR0)

    def _zero(j, carry):
        buf[j, pl.ds(0, 16)] = zvec
        buf[j, pl.ds(16, 16)] = zvec
        return carry
    lax.fori_loop(0, NRL, _zero, 0)
    for i in range(NS):
        @pl.when(t < N_TILES - 1)
        def _():
            pltpu.sync_copy(buf.at[pl.ds(0, NR0)],
                            acc.at[i].at[pl.ds(start, NR0)])

        @pl.when(t == N_TILES - 1)
        def _():
            pltpu.sync_copy(buf, acc.at[i].at[pl.ds(start, NRL)])
    plsc.subcore_barrier()

    base = t * CP0
    nc = jnp.where(t == N_TILES - 1, CPL, CP0)
    for i in range(NS):
        @pl.when((d == 0) & (t < N_TILES - 1))
        def _():
            pltpu.sync_copy(ev_hbm.at[i].at[pl.ds(base, CP0)],
                            gidx.at[pl.ds(0, CP0)])
            pltpu.sync_copy(eu_hbm.at[i].at[pl.ds(base, CP0)],
                            sidx.at[pl.ds(0, CP0)])
            pltpu.sync_copy(s0_hbm.at[i].at[pl.ds(base, CP0)],
                            vals.at[pl.ds(0, CP0)])

        @pl.when((d == 0) & (t == N_TILES - 1))
        def _():
            pltpu.sync_copy(ev_hbm.at[i].at[pl.ds(base, CPL)], gidx)
            pltpu.sync_copy(eu_hbm.at[i].at[pl.ds(base, CPL)], sidx)
            pltpu.sync_copy(s0_hbm.at[i].at[pl.ds(base, CPL)], vals)

        @pl.when((d == 1) & (t < N_TILES - 1))
        def _():
            pltpu.sync_copy(eu_hbm.at[i].at[pl.ds(base, CP0)],
                            gidx.at[pl.ds(0, CP0)])
            pltpu.sync_copy(ev_hbm.at[i].at[pl.ds(base, CP0)],
                            sidx.at[pl.ds(0, CP0)])
            pltpu.sync_copy(s1_hbm.at[i].at[pl.ds(base, CP0)],
                            vals.at[pl.ds(0, CP0)])

        @pl.when((d == 1) & (t == N_TILES - 1))
        def _():
            pltpu.sync_copy(eu_hbm.at[i].at[pl.ds(base, CPL)], gidx)
            pltpu.sync_copy(ev_hbm.at[i].at[pl.ds(base, CPL)], sidx)
            pltpu.sync_copy(s1_hbm.at[i].at[pl.ds(base, CPL)], vals)

        table = t_hbm.at[d].at[i]
        pltpu.async_copy(table.at[gidx.at[0]], rows.at[0], sems.at[0])

        def _chunk(g, carry):
            b = lax.rem(g, 2)
            nb = lax.rem(g + 1, 2)

            @pl.when(g >= 1)
            def _():
                pltpu.make_async_copy(rows.at[nb],
                                      acc.at[i].at[sidx.at[g - 1]],
                                      ssems.at[nb]).wait()

            @pl.when(g + 1 < nc)
            def _():
                pltpu.async_copy(table.at[gidx.at[g + 1]], rows.at[nb],
                                 sems.at[nb])

            pltpu.make_async_copy(table.at[gidx.at[g]], rows.at[b],
                                  sems.at[b]).wait()

            def _scale(q, c2):
                svec = vals[g, pl.ds(q * 16, 16)]
                for l in range(16):
                    e = q * 16 + l
                    s = svec[l]
                    rows[b, e, pl.ds(0, 16)] = rows[b, e, pl.ds(0, 16)] * s
                    rows[b, e, pl.ds(16, 16)] = rows[b, e, pl.ds(16, 16)] * s
                return c2
            lax.fori_loop(0, C // 16, _scale, 0)

            pltpu.async_copy(rows.at[b], acc.at[i].at[sidx.at[g]],
                             ssems.at[b], add=True)
            return carry
        lax.fori_loop(0, nc, _chunk, 0)

        lastb = lax.rem(nc - 1, 2)
        pltpu.make_async_copy(rows.at[lastb], acc.at[i].at[sidx.at[0]],
                              ssems.at[lastb]).wait()

    plsc.subcore_barrier()

    for i in range(NS):
        @pl.when(t < N_TILES - 1)
        def _():
            pltpu.sync_copy(acc.at[i].at[pl.ds(start, NR0)],
                            buf.at[pl.ds(0, NR0)])

        @pl.when(t == N_TILES - 1)
        def _():
            pltpu.sync_copy(acc.at[i].at[pl.ds(start, NRL)], buf)

        def _relu(j, carry):
            buf[j, pl.ds(0, 16)] = jnp.maximum(buf[j, pl.ds(0, 16)], zvec)
            buf[j, pl.ds(16, 16)] = jnp.maximum(buf[j, pl.ds(16, 16)], zvec)
            return carry
        lax.fori_loop(0, nrows, _relu, 0)

        col = pl.ds(i * D_SUB, D_SUB)

        @pl.when((d == 0) & (t < N_TILES - 1))
        def _():
            pltpu.sync_copy(buf.at[pl.ds(0, NR0)],
                            outu_hbm.at[pl.ds(start, NR0), col])

        @pl.when((d == 0) & (t == N_TILES - 1))
        def _():
            pltpu.sync_copy(buf, outu_hbm.at[pl.ds(start, NRL), col])

        @pl.when((d == 1) & (t < N_TILES - 1))
        def _():
            pltpu.sync_copy(buf.at[pl.ds(0, NR0)],
                            outv_hbm.at[pl.ds(start, NR0), col])

        @pl.when((d == 1) & (t == N_TILES - 1))
        def _():
            pltpu.sync_copy(buf, outv_hbm.at[pl.ds(start, NRL), col])


def _chunked(a):
    return a.reshape(NS, CHUNKS, C)


def kernel(x_u, x_v, edge_u, edge_v, sup_vals, sup_t_vals, W):
    T = pl.pallas_call(
        _tc_matmul,
        grid=(2, NS),
        in_specs=[
            pl.BlockSpec((N_U, D_IN), lambda d, i: (0, 0)),
            pl.BlockSpec((N_V, D_IN), lambda d, i: (0, 0)),
            pl.BlockSpec((1, D_IN, D_SUB), lambda d, i: (i, 0, 0)),
        ],
        out_specs=pl.BlockSpec((1, 1, N_U, D_SUB), lambda d, i: (d, i, 0, 0)),
        out_shape=jax.ShapeDtypeStruct((2, NS, N_U, D_SUB), jnp.float32),
    )(x_u, x_v, jnp.transpose(W.reshape(D_IN, NS, D_SUB), (1, 0, 2)))

    eu = _chunked(edge_u)
    ev = _chunked(edge_v)
    s0 = _chunked(sup_vals)
    s1 = _chunked(sup_t_vals)

    u_out, v_out = _sc_aggregate(T, eu, ev, s0, s1)
    return (u_out, v_out)


# speedup vs baseline: 19.0983x; 1.2176x over previous
import functools
import jax
import jax.numpy as jnp
from jax import lax
from jax.experimental import pallas as pl
from jax.experimental.pallas import tpu as pltpu
from jax.experimental.pallas import tpu_sc as plsc

N_U = 10000
N_V = 10000
E = 320000
D_IN = 128
D_OUT = 128
NS = 4
D_SUB = D_OUT // NS
E_BLK = E // NS
C = 128
CHUNKS = E_BLK // C
N_TILES = 16
CP0 = 39
CPL = 40
NR0 = 624
NRL = 640


def _tc_matmul(xu_ref, xv_ref, w_ref, out_ref):
    d = pl.program_id(0)

    @pl.when(d == 0)
    def _():
        out_ref[0] = jnp.dot(xv_ref[...], w_ref[...],
                             preferred_element_type=jnp.float32)

    @pl.when(d == 1)
    def _():
        out_ref[0] = jnp.dot(xu_ref[...], w_ref[...],
                             preferred_element_type=jnp.float32)


_SC_MESH = plsc.VectorSubcoreMesh(core_axis_name="c", subcore_axis_name="s")


@functools.partial(
    pl.kernel,
    out_type=(jax.ShapeDtypeStruct((N_U, D_OUT), jnp.float32),
              jax.ShapeDtypeStruct((N_V, D_OUT), jnp.float32)),
    mesh=_SC_MESH,
    scratch_types=[
        pltpu.VMEM_SHARED((NS, N_U, D_SUB), jnp.float32),
        pltpu.VMEM((CPL, C), jnp.int32),
        pltpu.VMEM((CPL, C), jnp.int32),
        pltpu.VMEM((CPL, C), jnp.float32),
        pltpu.VMEM((2, C, D_SUB), jnp.float32),
        pltpu.VMEM((NRL, D_SUB), jnp.float32),
        pltpu.SemaphoreType.DMA((2,)),
        pltpu.SemaphoreType.DMA((2,)),
    ],
    compiler_params=pltpu.CompilerParams(use_tc_tiling_on_sc=False),
)
def _sc_aggregate(t_hbm, eu_hbm, ev_hbm, geu_hbm, gev_hbm, s0_hbm, s1_hbm,
                  outu_hbm, outv_hbm,
                  acc, gidx, sidx, vals, rows, buf, sems, ssems):
    d = lax.axis_index("c")
    t = lax.axis_index("s")

    zvec = jnp.zeros((16,), jnp.float32)
    start = t * NR0
    nrows = jnp.where(t == N_TILES - 1, NRL, NR0)

    def _zero(j, carry):
        buf[j, pl.ds(0, 16)] = zvec
        buf[j, pl.ds(16, 16)] = zvec
        return carry
    lax.fori_loop(0, NRL, _zero, 0)
    for i in range(NS):
        @pl.when(t < N_TILES - 1)
        def _():
            pltpu.sync_copy(buf.at[pl.ds(0, NR0)],
                            acc.at[i].at[pl.ds(start, NR0)])

        @pl.when(t == N_TILES - 1)
        def _():
            pltpu.sync_copy(buf, acc.at[i].at[pl.ds(start, NRL)])
    plsc.subcore_barrier()

    base = t * CP0
    nc = jnp.where(t == N_TILES - 1, CPL, CP0)
    for i in range(NS):
        @pl.when((d == 0) & (t < N_TILES - 1))
        def _():
            pltpu.sync_copy(gev_hbm.at[i].at[pl.ds(base, CP0)],
                            gidx.at[pl.ds(0, CP0)])
            pltpu.sync_copy(eu_hbm.at[i].at[pl.ds(base, CP0)],
                            sidx.at[pl.ds(0, CP0)])
            pltpu.sync_copy(s0_hbm.at[i].at[pl.ds(base, CP0)],
                            vals.at[pl.ds(0, CP0)])

        @pl.when((d == 0) & (t == N_TILES - 1))
        def _():
            pltpu.sync_copy(gev_hbm.at[i].at[pl.ds(base, CPL)], gidx)
            pltpu.sync_copy(eu_hbm.at[i].at[pl.ds(base, CPL)], sidx)
            pltpu.sync_copy(s0_hbm.at[i].at[pl.ds(base, CPL)], vals)

        @pl.when((d == 1) & (t < N_TILES - 1))
        def _():
            pltpu.sync_copy(geu_hbm.at[i].at[pl.ds(base, CP0)],
                            gidx.at[pl.ds(0, CP0)])
            pltpu.sync_copy(ev_hbm.at[i].at[pl.ds(base, CP0)],
                            sidx.at[pl.ds(0, CP0)])
            pltpu.sync_copy(s1_hbm.at[i].at[pl.ds(base, CP0)],
                            vals.at[pl.ds(0, CP0)])

        @pl.when((d == 1) & (t == N_TILES - 1))
        def _():
            pltpu.sync_copy(geu_hbm.at[i].at[pl.ds(base, CPL)], gidx)
            pltpu.sync_copy(ev_hbm.at[i].at[pl.ds(base, CPL)], sidx)
            pltpu.sync_copy(s1_hbm.at[i].at[pl.ds(base, CPL)], vals)

        table = t_hbm.at[d]
        pltpu.async_copy(table.at[gidx.at[0]], rows.at[0], sems.at[0])

        def _chunk(g, carry):
            b = lax.rem(g, 2)
            nb = lax.rem(g + 1, 2)

            @pl.when(g >= 1)
            def _():
                pltpu.make_async_copy(rows.at[nb],
                                      acc.at[i].at[sidx.at[g - 1]],
                                      ssems.at[nb]).wait()

            @pl.when(g + 1 < nc)
            def _():
                pltpu.async_copy(table.at[gidx.at[g + 1]], rows.at[nb],
                                 sems.at[nb])

            pltpu.make_async_copy(table.at[gidx.at[g]], rows.at[b],
                                  sems.at[b]).wait()

            def _scale(q, c2):
                svec = vals[g, pl.ds(q * 16, 16)]
                for l in range(16):
                    e = q * 16 + l
                    s = svec[l]
                    rows[b, e, pl.ds(0, 16)] = rows[b, e, pl.ds(0, 16)] * s
                    rows[b, e, pl.ds(16, 16)] = rows[b, e, pl.ds(16, 16)] * s
                return c2
            lax.fori_loop(0, C // 16, _scale, 0)

            pltpu.async_copy(rows.at[b], acc.at[i].at[sidx.at[g]],
                             ssems.at[b], add=True)
            return carry
        lax.fori_loop(0, nc, _chunk, 0)

        lastb = lax.rem(nc - 1, 2)
        pltpu.make_async_copy(rows.at[lastb], acc.at[i].at[sidx.at[0]],
                              ssems.at[lastb]).wait()

    plsc.subcore_barrier()

    for i in range(NS):
        @pl.when(t < N_TILES - 1)
        def _():
            pltpu.sync_copy(acc.at[i].at[pl.ds(start, NR0)],
                            buf.at[pl.ds(0, NR0)])

        @pl.when(t == N_TILES - 1)
        def _():
            pltpu.sync_copy(acc.at[i].at[pl.ds(start, NRL)], buf)

        def _relu(j, carry):
            buf[j, pl.ds(0, 16)] = jnp.maximum(buf[j, pl.ds(0, 16)], zvec)
            buf[j, pl.ds(16, 16)] = jnp.maximum(buf[j, pl.ds(16, 16)], zvec)
            return carry
        lax.fori_loop(0, nrows, _relu, 0)

        col = pl.ds(i * D_SUB, D_SUB)

        @pl.when((d == 0) & (t < N_TILES - 1))
        def _():
            pltpu.sync_copy(buf.at[pl.ds(0, NR0)],
                            outu_hbm.at[pl.ds(start, NR0), col])

        @pl.when((d == 0) & (t == N_TILES - 1))
        def _():
            pltpu.sync_copy(buf, outu_hbm.at[pl.ds(start, NRL), col])

        @pl.when((d == 1) & (t < N_TILES - 1))
        def _():
            pltpu.sync_copy(buf.at[pl.ds(0, NR0)],
                            outv_hbm.at[pl.ds(start, NR0), col])

        @pl.when((d == 1) & (t == N_TILES - 1))
        def _():
            pltpu.sync_copy(buf, outv_hbm.at[pl.ds(start, NRL), col])


def _chunked(a):
    return a.reshape(NS, CHUNKS, C)


def kernel(x_u, x_v, edge_u, edge_v, sup_vals, sup_t_vals, W):
    T = pl.pallas_call(
        _tc_matmul,
        grid=(2,),
        in_specs=[
            pl.BlockSpec((N_U, D_IN), lambda d: (0, 0)),
            pl.BlockSpec((N_V, D_IN), lambda d: (0, 0)),
            pl.BlockSpec((D_IN, D_OUT), lambda d: (0, 0)),
        ],
        out_specs=pl.BlockSpec((1, N_U, D_OUT), lambda d: (d, 0, 0)),
        out_shape=jax.ShapeDtypeStruct((2, N_U, D_OUT), jnp.float32),
    )(x_u, x_v, W)
    T = T.reshape(2, N_U * NS, D_SUB)

    eu = _chunked(edge_u)
    ev = _chunked(edge_v)
    s0 = _chunked(sup_vals)
    s1 = _chunked(sup_t_vals)
    slice_off = jnp.arange(NS, dtype=jnp.int32)[:, None, None]
    geu = NS * eu + slice_off
    gev = NS * ev + slice_off

    u_out, v_out = _sc_aggregate(T, eu, ev, geu, gev, s0, s1)
    return (u_out, v_out)


# speedup vs baseline: 20.3264x; 1.0643x over previous
import functools
import jax
import jax.numpy as jnp
from jax import lax
from jax.experimental import pallas as pl
from jax.experimental.pallas import tpu as pltpu
from jax.experimental.pallas import tpu_sc as plsc

N_U = 10000
N_V = 10000
E = 320000
D_IN = 128
D_OUT = 128
NS = 4
D_SUB = D_OUT // NS
E_BLK = E // NS
C = 128
CHUNKS = E_BLK // C
TOT = NS * CHUNKS
N_TILES = 16
CBIG = 157
CSML = 156
NR0 = 624
NRL = 640


def _tc_matmul(xu_ref, xv_ref, w_ref, out_ref):
    d = pl.program_id(0)

    @pl.when(d == 0)
    def _():
        out_ref[0] = jnp.dot(xv_ref[...], w_ref[...],
                             preferred_element_type=jnp.float32)

    @pl.when(d == 1)
    def _():
        out_ref[0] = jnp.dot(xu_ref[...], w_ref[...],
                             preferred_element_type=jnp.float32)


_SC_MESH = plsc.VectorSubcoreMesh(core_axis_name="c", subcore_axis_name="s")


@functools.partial(
    pl.kernel,
    out_type=(jax.ShapeDtypeStruct((N_U, D_OUT), jnp.float32),
              jax.ShapeDtypeStruct((N_V, D_OUT), jnp.float32)),
    mesh=_SC_MESH,
    scratch_types=[
        pltpu.VMEM_SHARED((NS, N_U, D_SUB), jnp.float32),
        pltpu.VMEM((3, C), jnp.int32),
        pltpu.VMEM((3, C), jnp.int32),
        pltpu.VMEM((3, C), jnp.float32),
        pltpu.VMEM((2, C, D_SUB), jnp.float32),
        pltpu.VMEM((NRL, D_SUB), jnp.float32),
        pltpu.SemaphoreType.DMA((2,)),
        pltpu.SemaphoreType.DMA((2,)),
        pltpu.SemaphoreType.DMA((3, 3)),
    ],
    compiler_params=pltpu.CompilerParams(use_tc_tiling_on_sc=False),
)
def _sc_aggregate(t_hbm, eu_hbm, ev_hbm, geu_hbm, gev_hbm, s0_hbm, s1_hbm,
                  outu_hbm, outv_hbm,
                  acc, gidx_a, sidx_a, vals_a, rows, buf,
                  sems, ssems, stsems):
    d = lax.axis_index("c")
    t = lax.axis_index("s")

    zvec = jnp.zeros((16,), jnp.float32)
    start = t * NR0
    nrows = jnp.where(t == N_TILES - 1, NRL, NR0)
    cstart = t * CSML + jnp.minimum(t, 4)
    nct = jnp.where(t < 4, CBIG, CSML)

    def _pref(g):
        slot = lax.rem(g, 3)
        row = cstart + g

        @pl.when(d == 0)
        def _():
            pltpu.async_copy(gev_hbm.at[row], gidx_a.at[slot],
                             stsems.at[slot, 0])
            pltpu.async_copy(eu_hbm.at[row], sidx_a.at[slot],
                             stsems.at[slot, 1])
            pltpu.async_copy(s0_hbm.at[row], vals_a.at[slot],
                             stsems.at[slot, 2])

        @pl.when(d == 1)
        def _():
            pltpu.async_copy(geu_hbm.at[row], gidx_a.at[slot],
                             stsems.at[slot, 0])
            pltpu.async_copy(ev_hbm.at[row], sidx_a.at[slot],
                             stsems.at[slot, 1])
            pltpu.async_copy(s1_hbm.at[row], vals_a.at[slot],
                             stsems.at[slot, 2])

    def _pref_wait(g):
        slot = lax.rem(g, 3)
        pltpu.make_async_copy(eu_hbm.at[0], gidx_a.at[slot],
                              stsems.at[slot, 0]).wait()
        pltpu.make_async_copy(eu_hbm.at[0], sidx_a.at[slot],
                              stsems.at[slot, 1]).wait()
        pltpu.make_async_copy(s0_hbm.at[0], vals_a.at[slot],
                              stsems.at[slot, 2]).wait()

    _pref(0)
    _pref(1)

    def _zero(j, carry):
        buf[j, pl.ds(0, 16)] = zvec
        buf[j, pl.ds(16, 16)] = zvec
        return carry
    lax.fori_loop(0, NRL, _zero, 0)
    for i in range(NS):
        @pl.when(t < N_TILES - 1)
        def _():
            pltpu.sync_copy(buf.at[pl.ds(0, NR0)],
                            acc.at[i].at[pl.ds(start, NR0)])

        @pl.when(t == N_TILES - 1)
        def _():
            pltpu.sync_copy(buf, acc.at[i].at[pl.ds(start, NRL)])
    plsc.subcore_barrier()

    table = t_hbm.at[d]
    _pref_wait(0)
    pltpu.async_copy(table.at[gidx_a.at[0]], rows.at[0], sems.at[0])

    def _chunk(g, carry):
        b = lax.rem(g, 2)
        nb = lax.rem(g + 1, 2)
        i = lax.div(cstart + g, CHUNKS)

        @pl.when(g >= 1)
        def _():
            pltpu.make_async_copy(rows.at[nb], acc.at[0].at[sidx_a.at[0]],
                                  ssems.at[nb]).wait()

        @pl.when(g + 2 < nct)
        def _():
            _pref(g + 2)

        @pl.when(g + 1 < nct)
        def _():
            _pref_wait(g + 1)
            pltpu.async_copy(table.at[gidx_a.at[lax.rem(g + 1, 3)]],
                             rows.at[nb], sems.at[nb])

        pltpu.make_async_copy(table.at[gidx_a.at[0]], rows.at[b],
                              sems.at[b]).wait()

        slot = lax.rem(g, 3)

        def _scale(q, c2):
            svec = vals_a[slot, pl.ds(q * 16, 16)]
            for l in range(16):
                e = q * 16 + l
                s = svec[l]
                rows[b, e, pl.ds(0, 16)] = rows[b, e, pl.ds(0, 16)] * s
                rows[b, e, pl.ds(16, 16)] = rows[b, e, pl.ds(16, 16)] * s
            return c2
        lax.fori_loop(0, C // 16, _scale, 0)

        pltpu.async_copy(rows.at[b], acc.at[i].at[sidx_a.at[slot]],
                         ssems.at[b], add=True)
        return carry
    lax.fori_loop(0, nct, _chunk, 0)

    lastb = lax.rem(nct - 1, 2)
    pltpu.make_async_copy(rows.at[lastb], acc.at[0].at[sidx_a.at[0]],
                          ssems.at[lastb]).wait()

    plsc.subcore_barrier()

    for i in range(NS):
        @pl.when(t < N_TILES - 1)
        def _():
            pltpu.sync_copy(acc.at[i].at[pl.ds(start, NR0)],
                            buf.at[pl.ds(0, NR0)])

        @pl.when(t == N_TILES - 1)
        def _():
            pltpu.sync_copy(acc.at[i].at[pl.ds(start, NRL)], buf)

        def _relu(j, carry):
            buf[j, pl.ds(0, 16)] = jnp.maximum(buf[j, pl.ds(0, 16)], zvec)
            buf[j, pl.ds(16, 16)] = jnp.maximum(buf[j, pl.ds(16, 16)], zvec)
            return carry
        lax.fori_loop(0, nrows, _relu, 0)

        col = pl.ds(i * D_SUB, D_SUB)

        @pl.when((d == 0) & (t < N_TILES - 1))
        def _():
            pltpu.sync_copy(buf.at[pl.ds(0, NR0)],
                            outu_hbm.at[pl.ds(start, NR0), col])

        @pl.when((d == 0) & (t == N_TILES - 1))
        def _():
            pltpu.sync_copy(buf, outu_hbm.at[pl.ds(start, NRL), col])

        @pl.when((d == 1) & (t < N_TILES - 1))
        def _():
            pltpu.sync_copy(buf.at[pl.ds(0, NR0)],
                            outv_hbm.at[pl.ds(start, NR0), col])

        @pl.when((d == 1) & (t == N_TILES - 1))
        def _():
            pltpu.sync_copy(buf, outv_hbm.at[pl.ds(start, NRL), col])


def _chunked(a):
    return a.reshape(TOT, C)


def kernel(x_u, x_v, edge_u, edge_v, sup_vals, sup_t_vals, W):
    T = pl.pallas_call(
        _tc_matmul,
        grid=(2,),
        in_specs=[
            pl.BlockSpec((N_U, D_IN), lambda d: (0, 0)),
            pl.BlockSpec((N_V, D_IN), lambda d: (0, 0)),
            pl.BlockSpec((D_IN, D_OUT), lambda d: (0, 0)),
        ],
        out_specs=pl.BlockSpec((1, N_U, D_OUT), lambda d: (d, 0, 0)),
        out_shape=jax.ShapeDtypeStruct((2, N_U, D_OUT), jnp.float32),
    )(x_u, x_v, W)
    T = T.reshape(2, N_U * NS, D_SUB)

    eu = _chunked(edge_u)
    ev = _chunked(edge_v)
    s0 = _chunked(sup_vals)
    s1 = _chunked(sup_t_vals)
    slice_off = (jnp.arange(TOT, dtype=jnp.int32) // CHUNKS)[:, None]
    geu = NS * eu + slice_off
    gev = NS * ev + slice_off

    u_out, v_out = _sc_aggregate(T, eu, ev, geu, gev, s0, s1)
    return (u_out, v_out)
